# baseline (speedup 1.0000x reference)
import functools

import jax
import jax.numpy as jnp
from jax import lax
from jax.experimental import pallas as pl
from jax.experimental.pallas import tpu as pltpu
from jax.experimental.pallas import tpu_sc as plsc

BATCH = 32
N_CLS = 100
CTX_DIM = 512
SEQ_LEN = 77
N_CTX = 12
TP = 3
POOL = 1000
SUF = SEQ_LEN - 1 - N_CTX * TP
NC_SUF = SEQ_LEN - 1 - N_CTX
CTX_ROWS = N_CTX * TP

CB = 50
NCB = N_CLS // CB
PB = 125

_GW = 12
_RPW = 8
_D = N_CTX * CTX_DIM


def _sc_gather(table, idx):
    mesh = plsc.VectorSubcoreMesh(core_axis_name="c", subcore_axis_name="s")

    @functools.partial(
        pl.kernel,
        mesh=mesh,
        out_type=jax.ShapeDtypeStruct((_GW * _RPW, _D), jnp.float32),
        scratch_types=[
            pltpu.VMEM((_RPW,), jnp.int32),
            pltpu.VMEM((_RPW, _D), jnp.float32),
            pltpu.SemaphoreType.DMA,
        ],
    )
    def k(table_hbm, idx_hbm, out_hbm, idx_v, rows_v, sem):
        wid = lax.axis_index("s") * 2 + lax.axis_index("c")

        @pl.when(wid < _GW)
        def _():
            base = wid * _RPW
            pltpu.sync_copy(idx_hbm.at[pl.ds(base, _RPW)], idx_v)
            pltpu.async_copy(table_hbm.at[idx_v], rows_v, sem).wait()
            pltpu.sync_copy(rows_v, out_hbm.at[pl.ds(base, _RPW)])

    return k(table, idx)


_NCW = 25
_NCR = POOL // _NCW
_MID = N_CTX * CTX_DIM
_SUFW = NC_SUF * CTX_DIM
_ROWW = SEQ_LEN * CTX_DIM


def _sc_nc(tp2d, pre1, suf1, tok1):
    mesh = plsc.VectorSubcoreMesh(core_axis_name="c", subcore_axis_name="s")

    @functools.partial(
        pl.kernel,
        mesh=mesh,
        out_type=[
            jax.ShapeDtypeStruct((POOL, _ROWW), jnp.float32),
            jax.ShapeDtypeStruct((POOL, SEQ_LEN), jnp.int32),
        ],
        scratch_types=[
            pltpu.VMEM((1, CTX_DIM), jnp.float32),
            pltpu.VMEM((1, _SUFW), jnp.float32),
            pltpu.VMEM((1, SEQ_LEN), jnp.int32),
            pltpu.SemaphoreType.DMA,
        ],
    )
    def k(tp_hbm, pre_hbm, suf_hbm, tok_hbm, out_hbm, otok_hbm,
          pre_v, suf_v, tok_v, sem):
        wid = lax.axis_index("s") * 2 + lax.axis_index("c")

        @pl.when(wid < _NCW)
        def _():
            pltpu.sync_copy(pre_hbm, pre_v)
            pltpu.sync_copy(suf_hbm, suf_v)
            pltpu.sync_copy(tok_hbm, tok_v)
            base = wid * _NCR
            for g in range(0, _NCR, 8):
                handles = []
                for i in range(g, g + 8):
                    r = base + i
                    handles.append(pltpu.async_copy(
                        pre_v, out_hbm.at[pl.ds(r, 1), pl.ds(0, CTX_DIM)], sem))
                    handles.append(pltpu.async_copy(
                        tp_hbm.at[pl.ds(r, 1)],
                        out_hbm.at[pl.ds(r, 1), pl.ds(CTX_DIM, _MID)], sem))
                    handles.append(pltpu.async_copy(
                        suf_v,
                        out_hbm.at[pl.ds(r, 1), pl.ds(CTX_DIM + _MID, _SUFW)],
                        sem))
                    handles.append(pltpu.async_copy(
                        tok_v, otok_hbm.at[pl.ds(r, 1)], sem))
                for h in handles:
                    h.wait()

    return k(tp2d, pre1, suf1, tok1)


def _prompts_body(ctx_ref, pre_ref, suf_ref, tokp_ref, out_ref, tok_ref):
    out_ref[:, 0:1, :] = pre_ref[...]
    out_ref[:, 1:1 + CTX_ROWS, :] = jnp.broadcast_to(
        ctx_ref[...], (CB, CTX_ROWS, CTX_DIM))
    out_ref[:, 1 + CTX_ROWS:SEQ_LEN, :] = suf_ref[...]
    tok_ref[...] = tokp_ref[...]


def _assemble_prompts(ctx, token_prefix, token_suffix, tokp3):
    return pl.pallas_call(
        _prompts_body,
        grid=(NCB, BATCH),
        in_specs=[
            pl.BlockSpec((1, CTX_ROWS, CTX_DIM), lambda c, b: (b, 0, 0)),
            pl.BlockSpec((CB, 1, CTX_DIM), lambda c, b: (c, 0, 0)),
            pl.BlockSpec((CB, SUF, CTX_DIM), lambda c, b: (c, 0, 0)),
            pl.BlockSpec((CB, 1, SEQ_LEN), lambda c, b: (c, 0, 0)),
        ],
        out_specs=[
            pl.BlockSpec((CB, SEQ_LEN, CTX_DIM), lambda c, b: (b * NCB + c, 0, 0)),
            pl.BlockSpec((CB, 1, SEQ_LEN), lambda c, b: (b * NCB + c, 0, 0)),
        ],
        out_shape=[
            jax.ShapeDtypeStruct((BATCH * N_CLS, SEQ_LEN, CTX_DIM), jnp.float32),
            jax.ShapeDtypeStruct((BATCH * N_CLS, 1, SEQ_LEN), jnp.int32),
        ],
    )(ctx, token_prefix, token_suffix, tokp3)


def kernel(indices, text_prompt, token_prefix, token_suffix, tokenized_prompts,
           nc_token_prefix, nc_token_suffix, nc_tokenized_prompts):
    idx = indices.reshape(-1).astype(jnp.int32)
    table = text_prompt.reshape(POOL, _D)
    ctx = _sc_gather(table, idx).reshape(BATCH, CTX_ROWS, CTX_DIM)

    tokp3 = tokenized_prompts.reshape(N_CLS, 1, SEQ_LEN)
    prompts, tok3 = _assemble_prompts(ctx, token_prefix, token_suffix, tokp3)

    ncp2d, nc_tok = _sc_nc(
        table,
        nc_token_prefix.reshape(1, CTX_DIM),
        nc_token_suffix.reshape(1, _SUFW),
        nc_tokenized_prompts.reshape(1, SEQ_LEN))

    return (prompts,
            tok3.reshape(BATCH * N_CLS, SEQ_LEN),
            ncp2d.reshape(POOL, SEQ_LEN, CTX_DIM),
            nc_tok)


# speedup vs baseline: 2.6470x; 2.6470x over previous
import functools

import jax
import jax.numpy as jnp
from jax import lax
from jax.experimental import pallas as pl
from jax.experimental.pallas import tpu as pltpu
from jax.experimental.pallas import tpu_sc as plsc

BATCH = 32
N_CLS = 100
CTX_DIM = 512
SEQ_LEN = 77
N_CTX = 12
TP = 3
POOL = 1000
SUF = SEQ_LEN - 1 - N_CTX * TP
NC_SUF = SEQ_LEN - 1 - N_CTX
CTX_ROWS = N_CTX * TP
BN = BATCH * N_CLS
NIDX = BATCH * TP
CTXJ_ROWS = N_CTX * NIDX


def _sc_gather(table, idx):
    mesh = plsc.VectorSubcoreMesh(core_axis_name="c", subcore_axis_name="s")

    @functools.partial(
        pl.kernel,
        mesh=mesh,
        out_type=jax.ShapeDtypeStruct((CTXJ_ROWS, CTX_DIM), jnp.float32),
        scratch_types=[
            pltpu.VMEM((NIDX,), jnp.int32),
            pltpu.VMEM((NIDX, CTX_DIM), jnp.float32),
            pltpu.SemaphoreType.DMA,
        ],
    )
    def k(table_hbm, idx_hbm, out_hbm, idx_v, rows_v, sem):
        wid = lax.axis_index("s") * 2 + lax.axis_index("c")

        @pl.when(wid < N_CTX)
        def _():
            base = wid * NIDX
            pltpu.sync_copy(idx_hbm.at[pl.ds(base, NIDX)], idx_v)
            pltpu.async_copy(table_hbm.at[idx_v], rows_v, sem).wait()
            pltpu.sync_copy(rows_v, out_hbm.at[pl.ds(base, NIDX)])

    return k(table, idx)


def _sel_matmul(slab, k, sel_row):
    col = lax.broadcasted_iota(jnp.int32, (BN, k), 1)
    e = (col == sel_row).astype(jnp.float32)
    return jax.lax.dot_general(
        e, slab, (((1,), (0,)), ((), ())),
        precision=jax.lax.Precision.HIGHEST,
        preferred_element_type=jnp.float32)


def _prompts_body(ctxj_ref, pre_ref, suf_ref, out_ref):
    s = pl.program_id(0)
    i100 = lax.broadcasted_iota(jnp.int32, (BN, 1), 0) // N_CLS
    c100 = lax.broadcasted_iota(jnp.int32, (BN, 1), 0) % N_CLS

    @pl.when(s == 0)
    def _():
        out_ref[0] = _sel_matmul(pre_ref[0], N_CLS, c100)

    @pl.when((s >= 1) & (s < 1 + CTX_ROWS))
    def _():
        t = (s - 1) // N_CTX
        out_ref[0] = _sel_matmul(ctxj_ref[...], NIDX, TP * i100 + t)

    @pl.when(s >= 1 + CTX_ROWS)
    def _():
        out_ref[0] = _sel_matmul(suf_ref[0], N_CLS, c100)


def _assemble_prompts_t(ctxj, pre_t, suf_t):
    return pl.pallas_call(
        _prompts_body,
        grid=(SEQ_LEN,),
        in_specs=[
            pl.BlockSpec((NIDX, CTX_DIM),
                         lambda s: (jnp.clip(s - 1, 0, CTX_ROWS - 1) % N_CTX, 0)),
            pl.BlockSpec((1, N_CLS, CTX_DIM), lambda s: (0, 0, 0)),
            pl.BlockSpec((1, N_CLS, CTX_DIM),
                         lambda s: (jnp.clip(s - 1 - CTX_ROWS, 0, SUF - 1), 0, 0)),
        ],
        out_specs=pl.BlockSpec((1, BN, CTX_DIM), lambda s: (s, 0, 0)),
        out_shape=jax.ShapeDtypeStruct((SEQ_LEN, BN, CTX_DIM), jnp.float32),
    )(ctxj, pre_t, suf_t)


def _nc_body(tp_ref, pre_ref, suf_ref, out_ref):
    s = pl.program_id(0)

    @pl.when(s == 0)
    def _():
        out_ref[0] = jnp.broadcast_to(pre_ref[0], (POOL, CTX_DIM))

    @pl.when((s >= 1) & (s < 1 + N_CTX))
    def _():
        out_ref[0] = tp_ref[0]

    @pl.when(s >= 1 + N_CTX)
    def _():
        out_ref[0] = jnp.broadcast_to(suf_ref[0], (POOL, CTX_DIM))


def _assemble_nc_t(tp_t, ncpre_t, ncsuf_t):
    return pl.pallas_call(
        _nc_body,
        grid=(SEQ_LEN,),
        in_specs=[
            pl.BlockSpec((1, POOL, CTX_DIM),
                         lambda s: (jnp.clip(s - 1, 0, N_CTX - 1), 0, 0)),
            pl.BlockSpec((1, 1, CTX_DIM), lambda s: (0, 0, 0)),
            pl.BlockSpec((1, 1, CTX_DIM),
                         lambda s: (jnp.clip(s - 1 - N_CTX, 0, NC_SUF - 1), 0, 0)),
        ],
        out_specs=pl.BlockSpec((1, POOL, CTX_DIM), lambda s: (s, 0, 0)),
        out_shape=jax.ShapeDtypeStruct((SEQ_LEN, POOL, CTX_DIM), jnp.float32),
    )(tp_t, ncpre_t, ncsuf_t)


def _tok_body(tokp_ref, nctok_ref, tok_ref, nctok_out_ref):
    tok_ref[...] = jnp.concatenate([tokp_ref[...]] * BATCH, axis=1)
    nctok_out_ref[...] = jnp.broadcast_to(nctok_ref[...], (SEQ_LEN, POOL))


def _assemble_tok_t(tokp_t, nctok_t):
    return pl.pallas_call(
        _tok_body,
        out_shape=[
            jax.ShapeDtypeStruct((SEQ_LEN, BN), jnp.int32),
            jax.ShapeDtypeStruct((SEQ_LEN, POOL), jnp.int32),
        ],
    )(tokp_t, nctok_t)


def kernel(indices, text_prompt, token_prefix, token_suffix, tokenized_prompts,
           nc_token_prefix, nc_token_suffix, nc_tokenized_prompts):
    idx = indices.reshape(-1).astype(jnp.int32)
    idxall = (POOL * jnp.arange(N_CTX, dtype=jnp.int32)[:, None]
              + idx[None, :]).reshape(-1)

    tp_t = jnp.transpose(text_prompt, (1, 0, 2))
    table = tp_t.reshape(POOL * N_CTX, CTX_DIM)
    ctxj = _sc_gather(table, idxall)

    pre_t = jnp.transpose(token_prefix, (1, 0, 2))
    suf_t = jnp.transpose(token_suffix, (1, 0, 2))
    prompts_t = _assemble_prompts_t(ctxj, pre_t, suf_t)

    nc_t = _assemble_nc_t(tp_t,
                          jnp.transpose(nc_token_prefix, (1, 0, 2)),
                          jnp.transpose(nc_token_suffix, (1, 0, 2)))

    tok_t, nctok_t = _assemble_tok_t(tokenized_prompts.T,
                                     nc_tokenized_prompts.T)

    return (jnp.transpose(prompts_t, (1, 0, 2)),
            tok_t.T,
            jnp.transpose(nc_t, (1, 0, 2)),
            nctok_t.T)


# speedup vs baseline: 6.0134x; 2.2717x over previous
import functools

import jax
import jax.numpy as jnp
from jax import lax
from jax.experimental import pallas as pl
from jax.experimental.pallas import tpu as pltpu
from jax.experimental.pallas import tpu_sc as plsc

BATCH = 32
N_CLS = 100
CTX_DIM = 512
SEQ_LEN = 77
N_CTX = 12
TP = 3
POOL = 1000
SUF = SEQ_LEN - 1 - N_CTX * TP
NC_SUF = SEQ_LEN - 1 - N_CTX
CTX_ROWS = N_CTX * TP
BN = BATCH * N_CLS
NIDX = BATCH * TP
CTXJ_ROWS = N_CTX * NIDX


def _sc_gather(table, idx):
    mesh = plsc.VectorSubcoreMesh(core_axis_name="c", subcore_axis_name="s")

    @functools.partial(
        pl.kernel,
        mesh=mesh,
        out_type=jax.ShapeDtypeStruct((CTXJ_ROWS, CTX_DIM), jnp.float32),
        scratch_types=[
            pltpu.VMEM((NIDX,), jnp.int32),
            pltpu.VMEM((NIDX, CTX_DIM), jnp.float32),
            pltpu.SemaphoreType.DMA,
        ],
    )
    def k(table_hbm, idx_hbm, out_hbm, idx_v, rows_v, sem):
        wid = lax.axis_index("s") * 2 + lax.axis_index("c")

        @pl.when(wid < N_CTX)
        def _():
            base = wid * NIDX
            pltpu.sync_copy(idx_hbm.at[pl.ds(base, NIDX)], idx_v)
            pltpu.async_copy(table_hbm.at[idx_v], rows_v, sem).wait()
            pltpu.sync_copy(rows_v, out_hbm.at[pl.ds(base, NIDX)])

    return k(table, idx)


def _tile_class_plane(slab, out_ref):
    slab2 = jnp.concatenate([slab, slab], axis=0)
    for k in range(BN // (2 * N_CLS)):
        out_ref[0, 2 * N_CLS * k:2 * N_CLS * (k + 1), :] = slab2


def _prompts_body(ctxj_ref, pre_ref, suf_ref, out_ref):
    s = pl.program_id(0)

    @pl.when(s == 0)
    def _():
        _tile_class_plane(pre_ref[0], out_ref)

    @pl.when((s >= 1) & (s < 1 + CTX_ROWS))
    def _():
        t = (s - 1) // N_CTX
        for b in range(BATCH):
            row = ctxj_ref[pl.ds(TP * b + t, 1), :]
            out_ref[0, N_CLS * b:N_CLS * (b + 1), :] = jnp.broadcast_to(
                row, (N_CLS, CTX_DIM))

    @pl.when(s >= 1 + CTX_ROWS)
    def _():
        _tile_class_plane(suf_ref[0], out_ref)


def _assemble_prompts_t(ctxj, pre_t, suf_t):
    return pl.pallas_call(
        _prompts_body,
        grid=(SEQ_LEN,),
        in_specs=[
            pl.BlockSpec((NIDX, CTX_DIM),
                         lambda s: (jnp.clip(s - 1, 0, CTX_ROWS - 1) % N_CTX, 0)),
            pl.BlockSpec((1, N_CLS, CTX_DIM), lambda s: (0, 0, 0)),
            pl.BlockSpec((1, N_CLS, CTX_DIM),
                         lambda s: (jnp.clip(s - 1 - CTX_ROWS, 0, SUF - 1), 0, 0)),
        ],
        out_specs=pl.BlockSpec((1, BN, CTX_DIM), lambda s: (s, 0, 0)),
        out_shape=jax.ShapeDtypeStruct((SEQ_LEN, BN, CTX_DIM), jnp.float32),
    )(ctxj, pre_t, suf_t)


def _nc_body(tp_ref, pre_ref, suf_ref, out_ref):
    s = pl.program_id(0)

    @pl.when(s == 0)
    def _():
        out_ref[0] = jnp.broadcast_to(pre_ref[0], (POOL, CTX_DIM))

    @pl.when((s >= 1) & (s < 1 + N_CTX))
    def _():
        out_ref[0] = tp_ref[0]

    @pl.when(s >= 1 + N_CTX)
    def _():
        out_ref[0] = jnp.broadcast_to(suf_ref[0], (POOL, CTX_DIM))


def _assemble_nc_t(tp_t, ncpre_t, ncsuf_t):
    return pl.pallas_call(
        _nc_body,
        grid=(SEQ_LEN,),
        in_specs=[
            pl.BlockSpec((1, POOL, CTX_DIM),
                         lambda s: (jnp.clip(s - 1, 0, N_CTX - 1), 0, 0)),
            pl.BlockSpec((1, 1, CTX_DIM), lambda s: (0, 0, 0)),
            pl.BlockSpec((1, 1, CTX_DIM),
                         lambda s: (jnp.clip(s - 1 - N_CTX, 0, NC_SUF - 1), 0, 0)),
        ],
        out_specs=pl.BlockSpec((1, POOL, CTX_DIM), lambda s: (s, 0, 0)),
        out_shape=jax.ShapeDtypeStruct((SEQ_LEN, POOL, CTX_DIM), jnp.float32),
    )(tp_t, ncpre_t, ncsuf_t)


def _tok_body(tokp_ref, nctok_ref, tok_ref, nctok_out_ref):
    tok_ref[...] = jnp.concatenate([tokp_ref[...]] * BATCH, axis=1)
    nctok_out_ref[...] = jnp.broadcast_to(nctok_ref[...], (SEQ_LEN, POOL))


def _assemble_tok_t(tokp_t, nctok_t):
    return pl.pallas_call(
        _tok_body,
        out_shape=[
            jax.ShapeDtypeStruct((SEQ_LEN, BN), jnp.int32),
            jax.ShapeDtypeStruct((SEQ_LEN, POOL), jnp.int32),
        ],
    )(tokp_t, nctok_t)


def kernel(indices, text_prompt, token_prefix, token_suffix, tokenized_prompts,
           nc_token_prefix, nc_token_suffix, nc_tokenized_prompts):
    idx = indices.reshape(-1).astype(jnp.int32)
    idxall = (POOL * jnp.arange(N_CTX, dtype=jnp.int32)[:, None]
              + idx[None, :]).reshape(-1)

    tp_t = jnp.transpose(text_prompt, (1, 0, 2))
    table = tp_t.reshape(POOL * N_CTX, CTX_DIM)
    ctxj = _sc_gather(table, idxall)

    pre_t = jnp.transpose(token_prefix, (1, 0, 2))
    suf_t = jnp.transpose(token_suffix, (1, 0, 2))
    prompts_t = _assemble_prompts_t(ctxj, pre_t, suf_t)

    nc_t = _assemble_nc_t(tp_t,
                          jnp.transpose(nc_token_prefix, (1, 0, 2)),
                          jnp.transpose(nc_token_suffix, (1, 0, 2)))

    tok_t, nctok_t = _assemble_tok_t(tokenized_prompts.T,
                                     nc_tokenized_prompts.T)

    return (jnp.transpose(prompts_t, (1, 0, 2)),
            tok_t.T,
            jnp.transpose(nc_t, (1, 0, 2)),
            nctok_t.T)


# speedup vs baseline: 6.4489x; 1.0724x over previous
import functools

import jax
import jax.numpy as jnp
from jax import lax
from jax.experimental import pallas as pl
from jax.experimental.pallas import tpu as pltpu
from jax.experimental.pallas import tpu_sc as plsc

BATCH = 32
N_CLS = 100
CTX_DIM = 512
SEQ_LEN = 77
N_CTX = 12
TP = 3
POOL = 1000
SUF = SEQ_LEN - 1 - N_CTX * TP
NC_SUF = SEQ_LEN - 1 - N_CTX
CTX_ROWS = N_CTX * TP
BN = BATCH * N_CLS
NIDX = BATCH * TP
CTXJ_ROWS = N_CTX * NIDX


def _sc_gather(table, idx):
    mesh = plsc.VectorSubcoreMesh(core_axis_name="c", subcore_axis_name="s")

    @functools.partial(
        pl.kernel,
        mesh=mesh,
        out_type=jax.ShapeDtypeStruct((CTXJ_ROWS, CTX_DIM), jnp.float32),
        scratch_types=[
            pltpu.VMEM((NIDX,), jnp.int32),
            pltpu.VMEM((NIDX, CTX_DIM), jnp.float32),
            pltpu.SemaphoreType.DMA,
        ],
    )
    def k(table_hbm, idx_hbm, out_hbm, idx_v, rows_v, sem):
        wid = lax.axis_index("s") * 2 + lax.axis_index("c")

        @pl.when(wid < N_CTX)
        def _():
            base = wid * NIDX
            pltpu.sync_copy(idx_hbm.at[pl.ds(base, NIDX)], idx_v)
            pltpu.async_copy(table_hbm.at[idx_v], rows_v, sem).wait()
            pltpu.sync_copy(rows_v, out_hbm.at[pl.ds(base, NIDX)])

    return k(table, idx)


def _tile_class_plane(slab, out_ref):
    slab2 = jnp.concatenate([slab, slab], axis=0)
    for k in range(BN // (2 * N_CLS)):
        out_ref[0, 2 * N_CLS * k:2 * N_CLS * (k + 1), :] = slab2


def _prompts_body(ctxj_ref, pre_ref, suf_ref, out_ref):
    s = pl.program_id(0)

    @pl.when(s == 0)
    def _():
        _tile_class_plane(pre_ref[0], out_ref)

    @pl.when((s >= 1) & (s < 1 + CTX_ROWS))
    def _():
        t = (s - 1) // N_CTX
        for b in range(BATCH):
            row = ctxj_ref[pl.ds(TP * b + t, 1), :]
            out_ref[0, N_CLS * b:N_CLS * (b + 1), :] = jnp.broadcast_to(
                row, (N_CLS, CTX_DIM))

    @pl.when(s >= 1 + CTX_ROWS)
    def _():
        _tile_class_plane(suf_ref[0], out_ref)


def _assemble_prompts_t(ctxj, pre_t, suf_t):
    return pl.pallas_call(
        _prompts_body,
        grid=(SEQ_LEN,),
        in_specs=[
            pl.BlockSpec((NIDX, CTX_DIM),
                         lambda s: (jnp.clip(s - 1, 0, CTX_ROWS - 1) % N_CTX, 0)),
            pl.BlockSpec((1, N_CLS, CTX_DIM), lambda s: (0, 0, 0)),
            pl.BlockSpec((1, N_CLS, CTX_DIM),
                         lambda s: (jnp.clip(s - 1 - CTX_ROWS, 0, SUF - 1), 0, 0)),
        ],
        out_specs=pl.BlockSpec((1, BN, CTX_DIM), lambda s: (s, 0, 0)),
        out_shape=jax.ShapeDtypeStruct((SEQ_LEN, BN, CTX_DIM), jnp.float32),
    )(ctxj, pre_t, suf_t)


_NCP = 7


def _nc_body(tp_ref, pre_ref, suf_ref, out_ref):
    s = pl.program_id(0)
    for k in range(_NCP):
        sp = _NCP * s + k

        @pl.when(sp == 0)
        def _():
            out_ref[k] = jnp.broadcast_to(pre_ref[0], (POOL, CTX_DIM))

        @pl.when((sp >= 1) & (sp < 1 + N_CTX))
        def _():
            out_ref[k] = tp_ref[pl.ds(jnp.clip(sp - 1, 0, N_CTX - 1), 1)][0]

        @pl.when(sp >= 1 + N_CTX)
        def _():
            row = suf_ref[pl.ds(jnp.clip(sp - 1 - N_CTX, 0, NC_SUF - 1), 1)][0]
            out_ref[k] = jnp.broadcast_to(row, (POOL, CTX_DIM))


def _assemble_nc_t(tp_t, ncpre_t, ncsuf_t):
    return pl.pallas_call(
        _nc_body,
        grid=(SEQ_LEN // _NCP,),
        in_specs=[
            pl.BlockSpec((N_CTX, POOL, CTX_DIM), lambda s: (0, 0, 0)),
            pl.BlockSpec((1, 1, CTX_DIM), lambda s: (0, 0, 0)),
            pl.BlockSpec((NC_SUF, 1, CTX_DIM), lambda s: (0, 0, 0)),
        ],
        out_specs=pl.BlockSpec((_NCP, POOL, CTX_DIM), lambda s: (s, 0, 0)),
        out_shape=jax.ShapeDtypeStruct((SEQ_LEN, POOL, CTX_DIM), jnp.float32),
    )(tp_t, ncpre_t, ncsuf_t)


def _tok_body(tokp_ref, nctok_ref, tok_ref, nctok_out_ref):
    tok_ref[...] = jnp.concatenate([tokp_ref[...]] * BATCH, axis=1)
    nctok_out_ref[...] = jnp.broadcast_to(nctok_ref[...], (SEQ_LEN, POOL))


def _assemble_tok_t(tokp_t, nctok_t):
    return pl.pallas_call(
        _tok_body,
        out_shape=[
            jax.ShapeDtypeStruct((SEQ_LEN, BN), jnp.int32),
            jax.ShapeDtypeStruct((SEQ_LEN, POOL), jnp.int32),
        ],
    )(tokp_t, nctok_t)


def kernel(indices, text_prompt, token_prefix, token_suffix, tokenized_prompts,
           nc_token_prefix, nc_token_suffix, nc_tokenized_prompts):
    idx = indices.reshape(-1).astype(jnp.int32)
    idxall = (POOL * jnp.arange(N_CTX, dtype=jnp.int32)[:, None]
              + idx[None, :]).reshape(-1)

    tp_t = jnp.transpose(text_prompt, (1, 0, 2))
    table = tp_t.reshape(POOL * N_CTX, CTX_DIM)
    ctxj = _sc_gather(table, idxall)

    pre_t = jnp.transpose(token_prefix, (1, 0, 2))
    suf_t = jnp.transpose(token_suffix, (1, 0, 2))
    prompts_t = _assemble_prompts_t(ctxj, pre_t, suf_t)

    nc_t = _assemble_nc_t(tp_t,
                          jnp.transpose(nc_token_prefix, (1, 0, 2)),
                          jnp.transpose(nc_token_suffix, (1, 0, 2)))

    tok_t, nctok_t = _assemble_tok_t(tokenized_prompts.T,
                                     nc_tokenized_prompts.T)

    return (jnp.transpose(prompts_t, (1, 0, 2)),
            tok_t.T,
            jnp.transpose(nc_t, (1, 0, 2)),
            nctok_t.T)
